# Initial kernel scaffold; baseline (speedup 1.0000x reference)
import jax
import jax.numpy as jnp
from jax.experimental import pallas as pl


def kernel(x, table):
    raise NotImplementedError("write your pallas kernel here")


# speedup vs baseline: 9.2925x; 9.2925x over previous
import functools

import jax
import jax.numpy as jnp
from jax import lax
from jax.experimental import pallas as pl
from jax.experimental.pallas import tpu as pltpu
from jax.experimental.pallas import tpu_sc as plsc

_VOCAB = 100000
_EMB = 64
_BATCH = 4096
_SEQ = 200
_SEQ_PAD = 208
_LANES = 16
_NW = 32
_B_PER_W = _BATCH // _NW
_G0 = 128
_G1 = _SEQ - _G0


def _body(x_hbm, table_hbm, out_hbm, idx_v, rows_v, out_v, t0_v, sem):
    wid = lax.axis_index("s") * 2 + lax.axis_index("c")
    base = wid * _B_PER_W

    pltpu.sync_copy(table_hbm.at[0], t0_v)
    idx_v[pl.ds(_SEQ_PAD - _LANES, _LANES)] = jnp.zeros((_LANES,), jnp.int32)

    def row_body(i, carry):
        row = base + i
        pltpu.sync_copy(x_hbm.at[pl.ds(row * _SEQ, _SEQ)], idx_v.at[pl.ds(0, _SEQ)])
        cp0 = pltpu.async_copy(
            table_hbm.at[idx_v.at[pl.ds(0, _G0)]], rows_v.at[pl.ds(0, _G0)], sem)
        cp1 = pltpu.async_copy(
            table_hbm.at[idx_v.at[pl.ds(_G0, _G1)]], rows_v.at[pl.ds(_G0, _G1)], sem)

        def cnt_body(k, acc):
            v = idx_v[pl.ds(k * _LANES, _LANES)]
            return acc + jnp.where(v == 0, jnp.full((_LANES,), 1, jnp.int32),
                                   jnp.zeros((_LANES,), jnp.int32))

        cnt = lax.fori_loop(0, _SEQ_PAD // _LANES, cnt_body,
                            jnp.zeros((_LANES,), jnp.int32))
        n_pad = jnp.int32(_SEQ - _SEQ_PAD)
        for l in range(_LANES):
            n_pad = n_pad + cnt[l]

        cp0.wait()
        cp1.wait()

        def acc_body(s, accs):
            a0, a1, a2, a3 = accs
            for u in range(4):
                r = s * 4 + u
                a0 = a0 + rows_v[r, pl.ds(0, _LANES)]
                a1 = a1 + rows_v[r, pl.ds(_LANES, _LANES)]
                a2 = a2 + rows_v[r, pl.ds(2 * _LANES, _LANES)]
                a3 = a3 + rows_v[r, pl.ds(3 * _LANES, _LANES)]
            return (a0, a1, a2, a3)

        z = jnp.zeros((_LANES,), jnp.float32)
        a0, a1, a2, a3 = lax.fori_loop(0, _SEQ // 4, acc_body, (z, z, z, z))

        npf = jnp.broadcast_to(n_pad.astype(jnp.float32), (_LANES,))
        inv = 1.0 / jnp.maximum(jnp.float32(_SEQ) - npf, 1.0)
        out_v[i, pl.ds(0, _LANES)] = (a0 - npf * t0_v[pl.ds(0, _LANES)]) * inv
        out_v[i, pl.ds(_LANES, _LANES)] = (a1 - npf * t0_v[pl.ds(_LANES, _LANES)]) * inv
        out_v[i, pl.ds(2 * _LANES, _LANES)] = (a2 - npf * t0_v[pl.ds(2 * _LANES, _LANES)]) * inv
        out_v[i, pl.ds(3 * _LANES, _LANES)] = (a3 - npf * t0_v[pl.ds(3 * _LANES, _LANES)]) * inv
        return carry

    lax.fori_loop(0, _B_PER_W, row_body, 0)
    pltpu.sync_copy(out_v, out_hbm.at[pl.ds(base, _B_PER_W)])


_sc_call = functools.partial(
    pl.kernel,
    out_type=jax.ShapeDtypeStruct((_BATCH, _EMB), jnp.float32),
    mesh=plsc.VectorSubcoreMesh(core_axis_name="c", subcore_axis_name="s"),
    compiler_params=pltpu.CompilerParams(use_tc_tiling_on_sc=False),
    scratch_types=[
        pltpu.VMEM((_SEQ_PAD,), jnp.int32),
        pltpu.VMEM((_SEQ, _EMB), jnp.float32),
        pltpu.VMEM((_B_PER_W, _EMB), jnp.float32),
        pltpu.VMEM((_EMB,), jnp.float32),
        pltpu.SemaphoreType.DMA,
    ],
)(_body)


def kernel(x, table):
    return _sc_call(x.astype(jnp.int32).reshape(-1), table)


# speedup vs baseline: 14.4815x; 1.5584x over previous
import functools

import jax
import jax.numpy as jnp
from jax import lax
from jax.experimental import pallas as pl
from jax.experimental.pallas import tpu as pltpu
from jax.experimental.pallas import tpu_sc as plsc

_VOCAB = 100000
_EMB = 64
_BATCH = 4096
_SEQ = 200
_SEQ_PAD = 208
_LANES = 16
_NW = 32
_B_PER_W = _BATCH // _NW
_G0 = 128
_G1 = _SEQ - _G0


def _fire(x_hbm, table_hbm, row, idx, buf, sem):
    pltpu.sync_copy(x_hbm.at[pl.ds(row * _SEQ, _SEQ)], idx.at[pl.ds(0, _SEQ)])
    pltpu.async_copy(table_hbm.at[idx.at[pl.ds(0, _G0)]],
                     buf.at[pl.ds(0, _G0)], sem)
    pltpu.async_copy(table_hbm.at[idx.at[pl.ds(_G0, _G1)]],
                     buf.at[pl.ds(_G0, _G1)], sem)


def _drain(table_hbm, idx, buf, sem):
    pltpu.make_async_copy(table_hbm.at[idx.at[pl.ds(0, _G0)]],
                          buf.at[pl.ds(0, _G0)], sem).wait()
    pltpu.make_async_copy(table_hbm.at[idx.at[pl.ds(_G0, _G1)]],
                          buf.at[pl.ds(_G0, _G1)], sem).wait()


def _consume(idx, buf, t0_v, out_v, i_out):
    zi = jnp.zeros((_LANES,), jnp.int32)
    oi = jnp.full((_LANES,), 1, jnp.int32)

    def cnt_body(k, acc):
        v = idx[pl.ds(k * _LANES, _LANES)]
        return acc + jnp.where(v == 0, oi, zi)

    cnt = lax.fori_loop(0, _SEQ_PAD // _LANES, cnt_body, zi)
    n_pad = jnp.int32(_SEQ - _SEQ_PAD)
    for l in range(_LANES):
        n_pad = n_pad + cnt[l]

    def acc_body(s, accs):
        accs = list(accs)
        for u in range(4):
            r = s * 4 + u
            h = (u % 2) * 4
            for j in range(4):
                accs[h + j] = accs[h + j] + buf[r, pl.ds(j * _LANES, _LANES)]
        return tuple(accs)

    z = jnp.zeros((_LANES,), jnp.float32)
    a = lax.fori_loop(0, _SEQ // 4, acc_body, (z,) * 8)

    npf = jnp.broadcast_to(n_pad.astype(jnp.float32), (_LANES,))
    inv = 1.0 / jnp.maximum(jnp.float32(_SEQ) - npf, 1.0)
    for j in range(4):
        s_j = a[j] + a[4 + j]
        out_v[i_out, pl.ds(j * _LANES, _LANES)] = (
            (s_j - npf * t0_v[pl.ds(j * _LANES, _LANES)]) * inv)


def _body(x_hbm, table_hbm, out_hbm,
          idx0, idx1, buf0, buf1, out_v, t0_v, semA, semB):
    wid = lax.axis_index("s") * 2 + lax.axis_index("c")
    base = wid * _B_PER_W

    pltpu.sync_copy(table_hbm.at[0], t0_v)
    idx0[pl.ds(_SEQ_PAD - _LANES, _LANES)] = jnp.zeros((_LANES,), jnp.int32)
    idx1[pl.ds(_SEQ_PAD - _LANES, _LANES)] = jnp.zeros((_LANES,), jnp.int32)

    _fire(x_hbm, table_hbm, base, idx0, buf0, semA)

    def pair_body(i, carry):
        r0 = 2 * i
        r1 = 2 * i + 1
        _fire(x_hbm, table_hbm, base + r1, idx1, buf1, semB)
        _drain(table_hbm, idx0, buf0, semA)
        _consume(idx0, buf0, t0_v, out_v, r0)
        rn = jnp.minimum(r0 + 2, _B_PER_W - 1)
        _fire(x_hbm, table_hbm, base + rn, idx0, buf0, semA)
        _drain(table_hbm, idx1, buf1, semB)
        _consume(idx1, buf1, t0_v, out_v, r1)
        return carry

    lax.fori_loop(0, _B_PER_W // 2, pair_body, 0)
    _drain(table_hbm, idx0, buf0, semA)

    pltpu.sync_copy(out_v, out_hbm.at[pl.ds(base, _B_PER_W)])


_sc_call = functools.partial(
    pl.kernel,
    out_type=jax.ShapeDtypeStruct((_BATCH, _EMB), jnp.float32),
    mesh=plsc.VectorSubcoreMesh(core_axis_name="c", subcore_axis_name="s"),
    compiler_params=pltpu.CompilerParams(use_tc_tiling_on_sc=False),
    scratch_types=[
        pltpu.VMEM((_SEQ_PAD,), jnp.int32),
        pltpu.VMEM((_SEQ_PAD,), jnp.int32),
        pltpu.VMEM((_SEQ, _EMB), jnp.float32),
        pltpu.VMEM((_SEQ, _EMB), jnp.float32),
        pltpu.VMEM((_B_PER_W, _EMB), jnp.float32),
        pltpu.VMEM((_EMB,), jnp.float32),
        pltpu.SemaphoreType.DMA,
        pltpu.SemaphoreType.DMA,
    ],
)(_body)


def kernel(x, table):
    return _sc_call(x.astype(jnp.int32).reshape(-1), table)


# speedup vs baseline: 20.8742x; 1.4414x over previous
import functools

import jax
import jax.numpy as jnp
from jax import lax
from jax.experimental import pallas as pl
from jax.experimental.pallas import tpu as pltpu
from jax.experimental.pallas import tpu_sc as plsc

_VOCAB = 100000
_EMB = 64
_BATCH = 4096
_SEQ = 200
_LANES = 16
_NW = 32
_B_PER_W = _BATCH // _NW
_G0 = 128
_G1 = _SEQ - _G0
_NBUF = 4


def _fire(table_hbm, idx_all, r, buf, sem):
    pltpu.async_copy(table_hbm.at[idx_all.at[pl.ds(r * _SEQ, _G0)]],
                     buf.at[pl.ds(0, _G0)], sem)
    pltpu.async_copy(table_hbm.at[idx_all.at[pl.ds(r * _SEQ + _G0, _G1)]],
                     buf.at[pl.ds(_G0, _G1)], sem)


def _drain(table_hbm, idx_all, r, buf, sem):
    pltpu.make_async_copy(table_hbm.at[idx_all.at[pl.ds(r * _SEQ, _G0)]],
                          buf.at[pl.ds(0, _G0)], sem).wait()
    pltpu.make_async_copy(table_hbm.at[idx_all.at[pl.ds(r * _SEQ + _G0, _G1)]],
                          buf.at[pl.ds(_G0, _G1)], sem).wait()


def _count_pads(idx_all, r):
    zi = jnp.zeros((_LANES,), jnp.int32)
    oi = jnp.full((_LANES,), 1, jnp.int32)
    base = r * _SEQ

    def cnt_body(k, acc):
        v = idx_all[pl.ds(base + k * _LANES, _LANES)]
        return acc + jnp.where(v == 0, oi, zi)

    cnt = lax.fori_loop(0, _SEQ // _LANES - 1, cnt_body, zi)
    v11 = idx_all[pl.ds(base + 176, _LANES)]
    cnt = cnt + jnp.where(v11 == 0, oi, zi)
    lane = lax.iota(jnp.int32, _LANES)
    vt = idx_all[pl.ds(base + 184, _LANES)]
    cnt = cnt + jnp.where((vt == 0) & (lane >= 8), oi, zi)
    n_pad = jnp.int32(0)
    for l in range(_LANES):
        n_pad = n_pad + cnt[l]
    return n_pad


def _consume(buf, n_pad, t0_v, out_v, i_out):
    def acc_body(s, accs):
        accs = list(accs)
        for u in range(8):
            r = s * 8 + u
            h = (u % 2) * 4
            for j in range(4):
                accs[h + j] = accs[h + j] + buf[r, pl.ds(j * _LANES, _LANES)]
        return tuple(accs)

    z = jnp.zeros((_LANES,), jnp.float32)
    a = lax.fori_loop(0, _SEQ // 8, acc_body, (z,) * 8)

    npf = jnp.broadcast_to(n_pad.astype(jnp.float32), (_LANES,))
    inv = 1.0 / jnp.maximum(jnp.float32(_SEQ) - npf, 1.0)
    for j in range(4):
        s_j = a[j] + a[4 + j]
        out_v[i_out, pl.ds(j * _LANES, _LANES)] = (
            (s_j - npf * t0_v[pl.ds(j * _LANES, _LANES)]) * inv)


def _body(x_hbm, table_hbm, out_hbm,
          idx_all, b0, b1, b2, b3, out_v, t0_v, s0, s1, s2, s3):
    bufs = (b0, b1, b2, b3)
    sems = (s0, s1, s2, s3)
    wid = lax.axis_index("s") * 2 + lax.axis_index("c")
    base = wid * _B_PER_W

    pltpu.sync_copy(table_hbm.at[0], t0_v)
    pltpu.sync_copy(x_hbm.at[pl.ds(base * _SEQ, _B_PER_W * _SEQ)], idx_all)

    for b in range(_NBUF - 1):
        _fire(table_hbm, idx_all, jnp.int32(b), bufs[b], sems[b])

    def quad_body(i, carry):
        for b in range(_NBUF):
            r = i * _NBUF + b
            rn = jnp.minimum(r + (_NBUF - 1), _B_PER_W - 1)
            _fire(table_hbm, idx_all, rn, bufs[(b + _NBUF - 1) % _NBUF],
                  sems[(b + _NBUF - 1) % _NBUF])
            n_pad = _count_pads(idx_all, r)
            _drain(table_hbm, idx_all, r, bufs[b], sems[b])
            _consume(bufs[b], n_pad, t0_v, out_v, r)
        return carry

    lax.fori_loop(0, _B_PER_W // _NBUF, quad_body, 0)
    last = jnp.int32(_B_PER_W - 1)
    for b in range(_NBUF - 1):
        _drain(table_hbm, idx_all, last, bufs[b], sems[b])

    pltpu.sync_copy(out_v, out_hbm.at[pl.ds(base, _B_PER_W)])


_sc_call = functools.partial(
    pl.kernel,
    out_type=jax.ShapeDtypeStruct((_BATCH, _EMB), jnp.float32),
    mesh=plsc.VectorSubcoreMesh(core_axis_name="c", subcore_axis_name="s"),
    compiler_params=pltpu.CompilerParams(use_tc_tiling_on_sc=False),
    scratch_types=[
        pltpu.VMEM((_B_PER_W * _SEQ,), jnp.int32),
        pltpu.VMEM((_SEQ, _EMB), jnp.float32),
        pltpu.VMEM((_SEQ, _EMB), jnp.float32),
        pltpu.VMEM((_SEQ, _EMB), jnp.float32),
        pltpu.VMEM((_SEQ, _EMB), jnp.float32),
        pltpu.VMEM((_B_PER_W, _EMB), jnp.float32),
        pltpu.VMEM((_EMB,), jnp.float32),
        pltpu.SemaphoreType.DMA,
        pltpu.SemaphoreType.DMA,
        pltpu.SemaphoreType.DMA,
        pltpu.SemaphoreType.DMA,
    ],
)(_body)


def kernel(x, table):
    return _sc_call(x.astype(jnp.int32).reshape(-1), table)
